# baseline (speedup 1.0000x reference)
import functools

import jax
import jax.numpy as jnp
from jax import lax
from jax.experimental import pallas as pl
from jax.experimental.pallas import tpu as pltpu
from jax.experimental.pallas import tpu_sc as plsc

_NUM_WORKERS = 32
_CHUNK = 256
_NBUF = 4
_TW = 1024


@functools.partial(jax.jit, static_argnums=(1, 2))
def _format_table(wt, v, d):
    grid = (v + _TW - 1) // _TW

    def body(x, o):
        o[:, 0:d] = jnp.transpose(x[:, 0:_TW // 2], (1, 0))
        o[:, d:2 * d] = jnp.transpose(x[:, _TW // 2:_TW], (1, 0))

    return pl.pallas_call(
        body,
        grid=(grid,),
        in_specs=[pl.BlockSpec((d, _TW), lambda g: (0, g))],
        out_specs=pl.BlockSpec((_TW // 2, 2 * d), lambda g: (g, 0)),
        out_shape=jax.ShapeDtypeStruct((grid * _TW // 2, 2 * d),
                                       jnp.float32),
    )(wt)


@functools.partial(jax.jit, static_argnums=(2, 3, 4))
def _emb(idx, table, n_total, d, n_per_w):
    n_chunks = n_per_w // _CHUNK
    n_outer = n_chunks // _NBUF
    mesh = plsc.VectorSubcoreMesh(core_axis_name="c", subcore_axis_name="s")

    @functools.partial(
        pl.kernel,
        out_type=jax.ShapeDtypeStruct((n_total, 2 * d), jnp.float32),
        mesh=mesh,
        scratch_types=[
            pltpu.VMEM((n_per_w,), jnp.int32),
            [pltpu.VMEM((_CHUNK, d), jnp.float32) for _ in range(_NBUF)],
            [pltpu.SemaphoreType.DMA for _ in range(_NBUF)],
            [pltpu.SemaphoreType.DMA for _ in range(_NBUF)],
        ],
        compiler_params=pltpu.CompilerParams(use_tc_tiling_on_sc=False),
    )
    def emb(idx_hbm, table_hbm, out_hbm, idx_all, rows, sg, so):
        wid = lax.axis_index("s") * 2 + lax.axis_index("c")
        base = wid * n_per_w

        pltpu.sync_copy(idx_hbm.at[pl.ds(base, n_per_w)], idx_all)

        def fire_gather(b, c):
            pltpu.async_copy(
                table_hbm.at[idx_all.at[pl.ds(c * _CHUNK, _CHUNK)]],
                rows[b], sg[b])

        def wait_gather(b, c):
            pltpu.make_async_copy(
                table_hbm.at[idx_all.at[pl.ds(c * _CHUNK, _CHUNK)]],
                rows[b], sg[b]).wait()

        def fire_out(b, c):
            pltpu.async_copy(
                rows[b],
                out_hbm.at[pl.ds(base + c * _CHUNK, _CHUNK), pl.ds(0, d)],
                so[b])

        def wait_out(b, c):
            pltpu.make_async_copy(
                rows[b],
                out_hbm.at[pl.ds(base + c * _CHUNK, _CHUNK), pl.ds(0, d)],
                so[b]).wait()

        for b in range(_NBUF):
            fire_gather(b, b)

        def outer(g, carry):
            c0 = g * _NBUF
            for b in range(_NBUF):
                wait_gather(b, c0 + b)
                fire_out(b, c0 + b)
            for b in range(_NBUF):
                c_next = c0 + b + _NBUF

                @pl.when(c_next < n_chunks)
                def _():
                    wait_out(b, c0 + b)
                    fire_gather(b, c_next)

            return carry

        lax.fori_loop(0, n_outer, outer, 0)

        for b in range(_NBUF):
            wait_out(b, n_chunks - _NBUF + b)

    return emb(idx, table)


def kernel(x, weight):
    b, s = x.shape
    v, d = weight.shape
    n = b * s
    tablec = _format_table(weight.T, v, d)
    table = tablec.reshape(tablec.shape[0] * 2, d)
    idx = x.reshape(n).astype(jnp.int32)
    q = idx % _TW
    idx = idx - q + (q % (_TW // 2)) * 2 + q // (_TW // 2)
    out = _emb(idx, table, n, d, n // _NUM_WORKERS)
    return out[:, :d].reshape(b, s, d)


# speedup vs baseline: 1.5303x; 1.5303x over previous
import functools

import jax
import jax.numpy as jnp
from jax import lax
from jax.experimental import pallas as pl
from jax.experimental.pallas import tpu as pltpu
from jax.experimental.pallas import tpu_sc as plsc

_NUM_WORKERS = 32
_CHUNK = 256
_NBUF = 4
_TW = 4096


@functools.partial(jax.jit, static_argnums=(1, 2))
def _format_table(wt, v, d):
    grid = (v + _TW - 1) // _TW

    def body(x, o):
        o[:, 0:d] = jnp.transpose(x[:, 0:_TW // 2], (1, 0))
        o[:, d:2 * d] = jnp.transpose(x[:, _TW // 2:_TW], (1, 0))

    return pl.pallas_call(
        body,
        grid=(grid,),
        in_specs=[pl.BlockSpec((d, _TW), lambda g: (0, g))],
        out_specs=pl.BlockSpec((_TW // 2, 2 * d), lambda g: (g, 0)),
        out_shape=jax.ShapeDtypeStruct((grid * _TW // 2, 2 * d),
                                       jnp.float32),
        compiler_params=pltpu.CompilerParams(
            dimension_semantics=("parallel",)),
    )(wt)


@functools.partial(jax.jit, static_argnums=(2, 3, 4))
def _emb(idx, table, n_total, d, n_per_w):
    n_chunks = n_per_w // _CHUNK
    n_outer = n_chunks // _NBUF
    mesh = plsc.VectorSubcoreMesh(core_axis_name="c", subcore_axis_name="s")

    @functools.partial(
        pl.kernel,
        out_type=jax.ShapeDtypeStruct((n_total, 2 * d), jnp.float32),
        mesh=mesh,
        scratch_types=[
            pltpu.VMEM((n_per_w,), jnp.int32),
            [pltpu.VMEM((_CHUNK, d), jnp.float32) for _ in range(_NBUF)],
            [pltpu.SemaphoreType.DMA for _ in range(_NBUF)],
            [pltpu.SemaphoreType.DMA for _ in range(_NBUF)],
        ],
        compiler_params=pltpu.CompilerParams(use_tc_tiling_on_sc=False),
    )
    def emb(idx_hbm, table_hbm, out_hbm, idx_all, rows, sg, so):
        wid = lax.axis_index("s") * 2 + lax.axis_index("c")
        base = wid * n_per_w

        pltpu.sync_copy(idx_hbm.at[pl.ds(base, n_per_w)], idx_all)

        def fire_gather(b, c):
            pltpu.async_copy(
                table_hbm.at[idx_all.at[pl.ds(c * _CHUNK, _CHUNK)]],
                rows[b], sg[b])

        def wait_gather(b, c):
            pltpu.make_async_copy(
                table_hbm.at[idx_all.at[pl.ds(c * _CHUNK, _CHUNK)]],
                rows[b], sg[b]).wait()

        def fire_out(b, c):
            pltpu.async_copy(
                rows[b],
                out_hbm.at[pl.ds(base + c * _CHUNK, _CHUNK), pl.ds(0, d)],
                so[b])

        def wait_out(b, c):
            pltpu.make_async_copy(
                rows[b],
                out_hbm.at[pl.ds(base + c * _CHUNK, _CHUNK), pl.ds(0, d)],
                so[b]).wait()

        for b in range(_NBUF):
            fire_gather(b, b)

        def outer(g, carry):
            c0 = g * _NBUF
            for b in range(_NBUF):
                wait_gather(b, c0 + b)
                fire_out(b, c0 + b)
            for b in range(_NBUF):
                c_next = c0 + b + _NBUF

                @pl.when(c_next < n_chunks)
                def _():
                    wait_out(b, c0 + b)
                    fire_gather(b, c_next)

            return carry

        lax.fori_loop(0, n_outer, outer, 0)

        for b in range(_NBUF):
            wait_out(b, n_chunks - _NBUF + b)

    return emb(idx, table)


def kernel(x, weight):
    b, s = x.shape
    v, d = weight.shape
    n = b * s
    tablec = _format_table(weight.T, v, d)
    table = tablec.reshape(tablec.shape[0] * 2, d)
    idx = x.reshape(n).astype(jnp.int32)
    q = idx % _TW
    idx = idx - q + (q % (_TW // 2)) * 2 + q // (_TW // 2)
    out = _emb(idx, table, n, d, n // _NUM_WORKERS)
    return out[:, :d].reshape(b, s, d)
